# Initial kernel scaffold; baseline (speedup 1.0000x reference)
import jax
import jax.numpy as jnp
from jax import lax
from jax.experimental import pallas as pl
from jax.experimental.pallas import tpu as pltpu
from jax.experimental.pallas import tpu_sc as plsc

N = 100000
E = 1600000
C_IN = 14
NC = 2
NS = 16
L = 16
CHUNK = 128
BLK = 2048
N_T = 100352
E_PAD = -(-E // (NC * NS * CHUNK)) * (NC * NS * CHUNK)


def _mesh():
    return plsc.VectorSubcoreMesh(
        core_axis_name="c", subcore_axis_name="s", num_cores=NC, num_subcores=NS
    )


def _sc_scratch():
    return [
        pltpu.VMEM((CHUNK,), jnp.int32),
        pltpu.VMEM((CHUNK,), jnp.int32),
        pltpu.VMEM((CHUNK, L), jnp.float32),
        pltpu.VMEM_SHARED((N_T, L), jnp.float32),
        pltpu.SemaphoreType.DMA,
    ]


def _seg0(xp, srcp, dstp, zeros):
    rpt = N_T // NS
    epc = E_PAD // NC
    ept = epc // NS
    nch = ept // CHUNK

    def body(xp_hbm, src_hbm, dst_hbm, z_hbm, out_hbm, idx_s, idx_d, rows, acc, sem):
        c = lax.axis_index("c")
        s = lax.axis_index("s")
        r0 = s * rpt
        pltpu.sync_copy(z_hbm.at[pl.ds(r0, rpt)], acc.at[pl.ds(r0, rpt)])
        plsc.subcore_barrier()
        base = c * epc + s * ept

        def step(i, carry):
            off = base + i * CHUNK
            pltpu.sync_copy(src_hbm.at[pl.ds(off, CHUNK)], idx_s)
            pltpu.sync_copy(dst_hbm.at[pl.ds(off, CHUNK)], idx_d)
            pltpu.async_copy(xp_hbm.at[idx_s], rows, sem).wait()
            pltpu.sync_copy(rows, acc.at[idx_d], add=True)
            return carry

        lax.fori_loop(0, nch, step, 0)
        plsc.subcore_barrier()
        pltpu.sync_copy(acc.at[pl.ds(r0, rpt)], out_hbm.at[pl.ds(c * N_T + r0, rpt)])

    return pl.kernel(
        body,
        out_type=jax.ShapeDtypeStruct((NC * N_T, L), jnp.float32),
        mesh=_mesh(),
        scratch_types=_sc_scratch(),
    )(xp, srcp, dstp, zeros)


def _seg1(h0v, srcp, dstp, zeros):
    rpt = N_T // NS
    ept = E_PAD // NS
    nch = ept // CHUNK

    def body(h_hbm, src_hbm, dst_hbm, z_hbm, out_hbm, idx_s, idx_d, rows, acc, sem):
        c = lax.axis_index("c")
        s = lax.axis_index("s")
        r0 = s * rpt
        pltpu.sync_copy(z_hbm.at[pl.ds(r0, rpt)], acc.at[pl.ds(r0, rpt)])
        plsc.subcore_barrier()
        base = s * ept

        def step(i, carry):
            off = base + i * CHUNK
            pltpu.sync_copy(src_hbm.at[pl.ds(off, CHUNK)], idx_s)
            pltpu.sync_copy(dst_hbm.at[pl.ds(off, CHUNK)], idx_d)
            for j in range(CHUNK // L):
                sl = pl.ds(j * L, L)
                idx_s[sl] = idx_s[sl] * 2 + c
            pltpu.async_copy(h_hbm.at[idx_s], rows, sem).wait()
            pltpu.sync_copy(rows, acc.at[idx_d], add=True)
            return carry

        lax.fori_loop(0, nch, step, 0)
        plsc.subcore_barrier()
        pltpu.sync_copy(acc.at[pl.ds(r0, rpt)], out_hbm.at[pl.ds(c * N_T + r0, rpt)])

    return pl.kernel(
        body,
        out_type=jax.ShapeDtypeStruct((NC * N_T, L), jnp.float32),
        mesh=_mesh(),
        scratch_types=_sc_scratch(),
    )(h0v, srcp, dstp, zeros)


def _dense0(p0, p1, xp, wl, wr):
    grid = N_T // BLK

    def body(p0_ref, p1_ref, x_ref, wl_ref, wr_ref, o_ref):
        agg = p0_ref[...] + p1_ref[...]
        o_ref[...] = jnp.maximum(
            jnp.dot(agg, wl_ref[...], preferred_element_type=jnp.float32)
            + jnp.dot(x_ref[...], wr_ref[...], preferred_element_type=jnp.float32),
            0.0,
        )

    return pl.pallas_call(
        body,
        grid=(grid,),
        in_specs=[
            pl.BlockSpec((BLK, L), lambda i: (i, 0)),
            pl.BlockSpec((BLK, L), lambda i: (i, 0)),
            pl.BlockSpec((BLK, L), lambda i: (i, 0)),
            pl.BlockSpec((L, 32), lambda i: (0, 0)),
            pl.BlockSpec((L, 32), lambda i: (0, 0)),
        ],
        out_specs=pl.BlockSpec((BLK, 32), lambda i: (i, 0)),
        out_shape=jax.ShapeDtypeStruct((N_T, 32), jnp.float32),
    )(p0, p1, xp, wl, wr)


def _dense1(s1a, s1b, h0, wla, wlb, wr, w11, b11, w12, b12):
    grid = N_T // BLK

    def body(a_ref, b_ref, h_ref, wla_ref, wlb_ref, wr_ref, w11_ref, b11_ref,
             w12_ref, b12_ref, o_ref, acc_ref):
        i = pl.program_id(0)
        h1 = jnp.maximum(
            jnp.dot(a_ref[...], wla_ref[...], preferred_element_type=jnp.float32)
            + jnp.dot(b_ref[...], wlb_ref[...], preferred_element_type=jnp.float32)
            + jnp.dot(h_ref[...], wr_ref[...], preferred_element_type=jnp.float32),
            0.0,
        )
        part = jnp.sum(h1, axis=0, keepdims=True)

        @pl.when(i == 0)
        def _():
            acc_ref[...] = part

        @pl.when(i > 0)
        def _():
            acc_ref[...] = acc_ref[...] + part

        @pl.when(i == grid - 1)
        def _():
            z = jnp.maximum(
                jnp.dot(acc_ref[...], w11_ref[...], preferred_element_type=jnp.float32)
                + b11_ref[...],
                0.0,
            )
            o_ref[...] = (
                jnp.dot(z, w12_ref[...], preferred_element_type=jnp.float32)
                + b12_ref[...]
            )

    return pl.pallas_call(
        body,
        grid=(grid,),
        in_specs=[
            pl.BlockSpec((BLK, L), lambda i: (i, 0)),
            pl.BlockSpec((BLK, L), lambda i: (i, 0)),
            pl.BlockSpec((BLK, 32), lambda i: (i, 0)),
            pl.BlockSpec((L, 32), lambda i: (0, 0)),
            pl.BlockSpec((L, 32), lambda i: (0, 0)),
            pl.BlockSpec((32, 32), lambda i: (0, 0)),
            pl.BlockSpec((32, L), lambda i: (0, 0)),
            pl.BlockSpec((1, L), lambda i: (0, 0)),
            pl.BlockSpec((L, 1), lambda i: (0, 0)),
            pl.BlockSpec((1, 1), lambda i: (0, 0)),
        ],
        out_specs=pl.BlockSpec((1, 1), lambda i: (0, 0)),
        out_shape=jax.ShapeDtypeStruct((1, 1), jnp.float32),
        scratch_shapes=[pltpu.VMEM((1, 32), jnp.float32)],
    )(s1a, s1b, h0, wla, wlb, wr, w11, b11, w12, b12)


def kernel(x, edge_index, Wl0, Wr0, Wl1, Wr1, W11, b11, W12, b12):
    f32 = jnp.float32
    xp = jnp.zeros((N_T, L), f32).at[:N, :C_IN].set(x.astype(f32))
    ei = edge_index.astype(jnp.int32)
    pad = jnp.full((E_PAD - E,), N, jnp.int32)
    srcp = jnp.concatenate([ei[0], pad])
    dstp = jnp.concatenate([ei[1], pad])
    zeros = jnp.zeros((N_T, L), f32)

    seg0 = _seg0(xp, srcp, dstp, zeros)
    wl0 = jnp.zeros((L, 32), f32).at[:C_IN].set(Wl0.astype(f32))
    wr0 = jnp.zeros((L, 32), f32).at[:C_IN].set(Wr0.astype(f32))
    h0 = _dense0(seg0[:N_T], seg0[N_T:], xp, wl0, wr0)

    h0v = h0.reshape(NC * N_T, L)
    seg1 = _seg1(h0v, srcp, dstp, zeros)

    out = _dense1(
        seg1[:N_T], seg1[N_T:], h0,
        Wl1[:L].astype(f32), Wl1[L:].astype(f32), Wr1.astype(f32),
        W11.astype(f32), b11.reshape(1, L).astype(f32),
        W12.astype(f32), b12.reshape(1, 1).astype(f32),
    )
    return out.reshape(1)


# speedup vs baseline: 5.8849x; 5.8849x over previous
import jax
import jax.numpy as jnp
from jax import lax
from jax.experimental import pallas as pl
from jax.experimental.pallas import tpu as pltpu
from jax.experimental.pallas import tpu_sc as plsc

N = 100000
E = 1600000
C_IN = 14
NC = 2
NS = 16
L = 16
CHUNK = 128
BLK = 2048
N_T = 100352
E_PAD = -(-E // (NC * NS * CHUNK)) * (NC * NS * CHUNK)


def _mesh():
    return plsc.VectorSubcoreMesh(
        core_axis_name="c", subcore_axis_name="s", num_cores=NC, num_subcores=NS
    )


_SC_PARAMS = pltpu.CompilerParams(use_tc_tiling_on_sc=False)


def _sc_scratch():
    return [
        pltpu.VMEM((CHUNK,), jnp.int32),
        pltpu.VMEM((CHUNK,), jnp.int32),
        pltpu.VMEM((CHUNK, L), jnp.float32),
        pltpu.VMEM_SHARED((N_T, L), jnp.float32),
        pltpu.SemaphoreType.DMA,
    ]


def _seg0(xp, srcp, dstp, zeros):
    rpt = N_T // NS
    epc = E_PAD // NC
    ept = epc // NS
    nch = ept // CHUNK

    def body(xp_hbm, src_hbm, dst_hbm, z_hbm, out_hbm, idx_s, idx_d, rows, acc, sem):
        c = lax.axis_index("c")
        s = lax.axis_index("s")
        r0 = s * rpt
        pltpu.sync_copy(z_hbm.at[pl.ds(r0, rpt)], acc.at[pl.ds(r0, rpt)])
        plsc.subcore_barrier()
        base = c * epc + s * ept

        def step(i, carry):
            off = base + i * CHUNK
            pltpu.sync_copy(src_hbm.at[pl.ds(off, CHUNK)], idx_s)
            pltpu.sync_copy(dst_hbm.at[pl.ds(off, CHUNK)], idx_d)
            pltpu.async_copy(xp_hbm.at[idx_s], rows, sem).wait()
            pltpu.sync_copy(rows, acc.at[idx_d], add=True)
            return carry

        lax.fori_loop(0, nch, step, 0)
        plsc.subcore_barrier()
        pltpu.sync_copy(acc.at[pl.ds(r0, rpt)], out_hbm.at[pl.ds(c * N_T + r0, rpt)])

    return pl.kernel(
        body,
        out_type=jax.ShapeDtypeStruct((NC * N_T, L), jnp.float32),
        mesh=_mesh(),
        scratch_types=_sc_scratch(),
        compiler_params=_SC_PARAMS,
    )(xp, srcp, dstp, zeros)


def _seg1(h0v, srcp, dstp, zeros):
    rpt = N_T // NS
    ept = E_PAD // NS
    nch = ept // CHUNK

    def body(h_hbm, src_hbm, dst_hbm, z_hbm, out_hbm, idx_s, idx_d, rows, acc, sem):
        c = lax.axis_index("c")
        s = lax.axis_index("s")
        r0 = s * rpt
        pltpu.sync_copy(z_hbm.at[pl.ds(r0, rpt)], acc.at[pl.ds(r0, rpt)])
        plsc.subcore_barrier()
        base = s * ept

        def step(i, carry):
            off = base + i * CHUNK
            pltpu.sync_copy(src_hbm.at[pl.ds(off, CHUNK)], idx_s)
            pltpu.sync_copy(dst_hbm.at[pl.ds(off, CHUNK)], idx_d)
            for j in range(CHUNK // L):
                sl = pl.ds(j * L, L)
                idx_s[sl] = idx_s[sl] * 2 + c
            pltpu.async_copy(h_hbm.at[idx_s], rows, sem).wait()
            pltpu.sync_copy(rows, acc.at[idx_d], add=True)
            return carry

        lax.fori_loop(0, nch, step, 0)
        plsc.subcore_barrier()
        pltpu.sync_copy(acc.at[pl.ds(r0, rpt)], out_hbm.at[pl.ds(c * N_T + r0, rpt)])

    return pl.kernel(
        body,
        out_type=jax.ShapeDtypeStruct((NC * N_T, L), jnp.float32),
        mesh=_mesh(),
        scratch_types=_sc_scratch(),
        compiler_params=_SC_PARAMS,
    )(h0v, srcp, dstp, zeros)


def _dense0(p0, p1, xp, wl, wr):
    grid = N_T // BLK

    def body(p0_ref, p1_ref, x_ref, wl_ref, wr_ref, o_ref):
        agg = p0_ref[...] + p1_ref[...]
        o_ref[...] = jnp.maximum(
            jnp.dot(agg, wl_ref[...], preferred_element_type=jnp.float32)
            + jnp.dot(x_ref[...], wr_ref[...], preferred_element_type=jnp.float32),
            0.0,
        )

    return pl.pallas_call(
        body,
        grid=(grid,),
        in_specs=[
            pl.BlockSpec((BLK, L), lambda i: (i, 0)),
            pl.BlockSpec((BLK, L), lambda i: (i, 0)),
            pl.BlockSpec((BLK, L), lambda i: (i, 0)),
            pl.BlockSpec((L, 32), lambda i: (0, 0)),
            pl.BlockSpec((L, 32), lambda i: (0, 0)),
        ],
        out_specs=pl.BlockSpec((BLK, 32), lambda i: (i, 0)),
        out_shape=jax.ShapeDtypeStruct((N_T, 32), jnp.float32),
    )(p0, p1, xp, wl, wr)


def _dense1(s1a, s1b, h0, wla, wlb, wr, w11, b11, w12, b12):
    grid = N_T // BLK

    def body(a_ref, b_ref, h_ref, wla_ref, wlb_ref, wr_ref, w11_ref, b11_ref,
             w12_ref, b12_ref, o_ref, acc_ref):
        i = pl.program_id(0)
        h1 = jnp.maximum(
            jnp.dot(a_ref[...], wla_ref[...], preferred_element_type=jnp.float32)
            + jnp.dot(b_ref[...], wlb_ref[...], preferred_element_type=jnp.float32)
            + jnp.dot(h_ref[...], wr_ref[...], preferred_element_type=jnp.float32),
            0.0,
        )
        part = jnp.sum(h1, axis=0, keepdims=True)

        @pl.when(i == 0)
        def _():
            acc_ref[...] = part

        @pl.when(i > 0)
        def _():
            acc_ref[...] = acc_ref[...] + part

        @pl.when(i == grid - 1)
        def _():
            z = jnp.maximum(
                jnp.dot(acc_ref[...], w11_ref[...], preferred_element_type=jnp.float32)
                + b11_ref[...],
                0.0,
            )
            o_ref[...] = (
                jnp.dot(z, w12_ref[...], preferred_element_type=jnp.float32)
                + b12_ref[...]
            )

    return pl.pallas_call(
        body,
        grid=(grid,),
        in_specs=[
            pl.BlockSpec((BLK, L), lambda i: (i, 0)),
            pl.BlockSpec((BLK, L), lambda i: (i, 0)),
            pl.BlockSpec((BLK, 32), lambda i: (i, 0)),
            pl.BlockSpec((L, 32), lambda i: (0, 0)),
            pl.BlockSpec((L, 32), lambda i: (0, 0)),
            pl.BlockSpec((32, 32), lambda i: (0, 0)),
            pl.BlockSpec((32, L), lambda i: (0, 0)),
            pl.BlockSpec((1, L), lambda i: (0, 0)),
            pl.BlockSpec((L, 1), lambda i: (0, 0)),
            pl.BlockSpec((1, 1), lambda i: (0, 0)),
        ],
        out_specs=pl.BlockSpec((1, 1), lambda i: (0, 0)),
        out_shape=jax.ShapeDtypeStruct((1, 1), jnp.float32),
        scratch_shapes=[pltpu.VMEM((1, 32), jnp.float32)],
    )(s1a, s1b, h0, wla, wlb, wr, w11, b11, w12, b12)


def kernel(x, edge_index, Wl0, Wr0, Wl1, Wr1, W11, b11, W12, b12):
    f32 = jnp.float32
    xp = jnp.zeros((N_T, L), f32).at[:N, :C_IN].set(x.astype(f32))
    ei = edge_index.astype(jnp.int32)
    pad = jnp.full((E_PAD - E,), N, jnp.int32)
    srcp = jnp.concatenate([ei[0], pad])
    dstp = jnp.concatenate([ei[1], pad])
    zeros = jnp.zeros((N_T, L), f32)

    seg0 = _seg0(xp, srcp, dstp, zeros)
    wl0 = jnp.zeros((L, 32), f32).at[:C_IN].set(Wl0.astype(f32))
    wr0 = jnp.zeros((L, 32), f32).at[:C_IN].set(Wr0.astype(f32))
    h0 = _dense0(seg0[:N_T], seg0[N_T:], xp, wl0, wr0)

    h0v = h0.reshape(NC * N_T, L)
    seg1 = _seg1(h0v, srcp, dstp, zeros)

    out = _dense1(
        seg1[:N_T], seg1[N_T:], h0,
        Wl1[:L].astype(f32), Wl1[L:].astype(f32), Wr1.astype(f32),
        W11.astype(f32), b11.reshape(1, L).astype(f32),
        W12.astype(f32), b12.reshape(1, 1).astype(f32),
    )
    return out.reshape(1)


# speedup vs baseline: 12.1824x; 2.0701x over previous
import jax
import jax.numpy as jnp
from jax import lax
from jax.experimental import pallas as pl
from jax.experimental.pallas import tpu as pltpu
from jax.experimental.pallas import tpu_sc as plsc

N = 100000
E = 1600000
C_IN = 14
NC = 2
NS = 16
L = 16
CHUNK = 128
BLK = 2048
N_T = 100352
E_PAD = -(-E // (NC * NS * CHUNK * 8)) * (NC * NS * CHUNK * 8)


def _mesh():
    return plsc.VectorSubcoreMesh(
        core_axis_name="c", subcore_axis_name="s", num_cores=NC, num_subcores=NS
    )


_SC_PARAMS = pltpu.CompilerParams(use_tc_tiling_on_sc=False)


K_CH = 8


def _segsum(tab, src2d, dst2d, zeros, feat_split):
    rpt = N_T // NS
    nch = (E_PAD // CHUNK) // NS if feat_split else (E_PAD // CHUNK) // (NC * NS)
    ngrp = nch // K_CH

    def body(tab_hbm, src_hbm, dst_hbm, z_hbm, out_hbm,
             idx_s, idx_d, rows, acc, sem_g, sem_s):
        c = lax.axis_index("c")
        s = lax.axis_index("s")
        r0 = s * rpt
        pltpu.sync_copy(z_hbm.at[pl.ds(r0, rpt)], acc.at[pl.ds(r0, rpt)])
        plsc.subcore_barrier()
        cbase = s * nch if feat_split else (c * NS + s) * nch

        def group(g, carry):
            row0 = cbase + g * K_CH
            pltpu.sync_copy(src_hbm.at[pl.ds(row0, K_CH)], idx_s)
            pltpu.sync_copy(dst_hbm.at[pl.ds(row0, K_CH)], idx_d)
            if feat_split:
                for j in range(K_CH):
                    for t in range(CHUNK // L):
                        sl = pl.ds(t * L, L)
                        idx_s[j, sl] = idx_s[j, sl] * 2 + c
            gds = [pltpu.async_copy(tab_hbm.at[idx_s.at[j]], rows.at[j], sem_g)
                   for j in range(K_CH)]
            for d in gds:
                d.wait()
            sds = [pltpu.async_copy(rows.at[j], acc.at[idx_d.at[j]], sem_s, add=True)
                   for j in range(K_CH)]
            for d in sds:
                d.wait()
            return carry

        lax.fori_loop(0, ngrp, group, 0)
        plsc.subcore_barrier()
        pltpu.sync_copy(acc.at[pl.ds(r0, rpt)], out_hbm.at[pl.ds(c * N_T + r0, rpt)])

    return pl.kernel(
        body,
        out_type=jax.ShapeDtypeStruct((NC * N_T, L), jnp.float32),
        mesh=_mesh(),
        scratch_types=[
            pltpu.VMEM((K_CH, CHUNK), jnp.int32),
            pltpu.VMEM((K_CH, CHUNK), jnp.int32),
            pltpu.VMEM((K_CH, CHUNK, L), jnp.float32),
            pltpu.VMEM_SHARED((N_T, L), jnp.float32),
            pltpu.SemaphoreType.DMA,
            pltpu.SemaphoreType.DMA,
        ],
        compiler_params=_SC_PARAMS,
    )(tab, src2d, dst2d, zeros)


def _dense0(p0, p1, xp, wl, wr):
    grid = N_T // BLK

    def body(p0_ref, p1_ref, x_ref, wl_ref, wr_ref, o_ref):
        agg = p0_ref[...] + p1_ref[...]
        o_ref[...] = jnp.maximum(
            jnp.dot(agg, wl_ref[...], preferred_element_type=jnp.float32, precision=lax.Precision.HIGHEST)
            + jnp.dot(x_ref[...], wr_ref[...], preferred_element_type=jnp.float32, precision=lax.Precision.HIGHEST),
            0.0,
        )

    return pl.pallas_call(
        body,
        grid=(grid,),
        in_specs=[
            pl.BlockSpec((BLK, L), lambda i: (i, 0)),
            pl.BlockSpec((BLK, L), lambda i: (i, 0)),
            pl.BlockSpec((BLK, L), lambda i: (i, 0)),
            pl.BlockSpec((L, 32), lambda i: (0, 0)),
            pl.BlockSpec((L, 32), lambda i: (0, 0)),
        ],
        out_specs=pl.BlockSpec((BLK, 32), lambda i: (i, 0)),
        out_shape=jax.ShapeDtypeStruct((N_T, 32), jnp.float32),
    )(p0, p1, xp, wl, wr)


def _dense1(s1a, s1b, h0, wla, wlb, wr, w11, b11, w12, b12):
    grid = N_T // BLK

    def body(a_ref, b_ref, h_ref, wla_ref, wlb_ref, wr_ref, w11_ref, b11_ref,
             w12_ref, b12_ref, o_ref, acc_ref):
        i = pl.program_id(0)
        h1 = jnp.maximum(
            jnp.dot(a_ref[...], wla_ref[...], preferred_element_type=jnp.float32, precision=lax.Precision.HIGHEST)
            + jnp.dot(b_ref[...], wlb_ref[...], preferred_element_type=jnp.float32, precision=lax.Precision.HIGHEST)
            + jnp.dot(h_ref[...], wr_ref[...], preferred_element_type=jnp.float32, precision=lax.Precision.HIGHEST),
            0.0,
        )
        part = jnp.sum(h1, axis=0, keepdims=True)

        @pl.when(i == 0)
        def _():
            acc_ref[...] = part

        @pl.when(i > 0)
        def _():
            acc_ref[...] = acc_ref[...] + part

        @pl.when(i == grid - 1)
        def _():
            z = jnp.maximum(
                jnp.dot(acc_ref[...], w11_ref[...], preferred_element_type=jnp.float32, precision=lax.Precision.HIGHEST)
                + b11_ref[...],
                0.0,
            )
            o_ref[...] = (
                jnp.dot(z, w12_ref[...], preferred_element_type=jnp.float32, precision=lax.Precision.HIGHEST)
                + b12_ref[...]
            )

    return pl.pallas_call(
        body,
        grid=(grid,),
        in_specs=[
            pl.BlockSpec((BLK, L), lambda i: (i, 0)),
            pl.BlockSpec((BLK, L), lambda i: (i, 0)),
            pl.BlockSpec((BLK, 32), lambda i: (i, 0)),
            pl.BlockSpec((L, 32), lambda i: (0, 0)),
            pl.BlockSpec((L, 32), lambda i: (0, 0)),
            pl.BlockSpec((32, 32), lambda i: (0, 0)),
            pl.BlockSpec((32, L), lambda i: (0, 0)),
            pl.BlockSpec((1, L), lambda i: (0, 0)),
            pl.BlockSpec((L, 1), lambda i: (0, 0)),
            pl.BlockSpec((1, 1), lambda i: (0, 0)),
        ],
        out_specs=pl.BlockSpec((1, 1), lambda i: (0, 0)),
        out_shape=jax.ShapeDtypeStruct((1, 1), jnp.float32),
        scratch_shapes=[pltpu.VMEM((1, 32), jnp.float32)],
    )(s1a, s1b, h0, wla, wlb, wr, w11, b11, w12, b12)


def kernel(x, edge_index, Wl0, Wr0, Wl1, Wr1, W11, b11, W12, b12):
    f32 = jnp.float32
    xp = jnp.zeros((N_T, L), f32).at[:N, :C_IN].set(x.astype(f32))
    ei = edge_index.astype(jnp.int32)
    pad = jnp.full((E_PAD - E,), N, jnp.int32)
    src2d = jnp.concatenate([ei[0], pad]).reshape(-1, CHUNK)
    dst2d = jnp.concatenate([ei[1], pad]).reshape(-1, CHUNK)
    zeros = jnp.zeros((N_T, L), f32)

    seg0 = _segsum(xp, src2d, dst2d, zeros, feat_split=False)
    wl0 = jnp.zeros((L, 32), f32).at[:C_IN].set(Wl0.astype(f32))
    wr0 = jnp.zeros((L, 32), f32).at[:C_IN].set(Wr0.astype(f32))
    h0 = _dense0(seg0[:N_T], seg0[N_T:], xp, wl0, wr0)

    h0v = h0.reshape(NC * N_T, L)
    seg1 = _segsum(h0v, src2d, dst2d, zeros, feat_split=True)

    out = _dense1(
        seg1[:N_T], seg1[N_T:], h0,
        Wl1[:L].astype(f32), Wl1[L:].astype(f32), Wr1.astype(f32),
        W11.astype(f32), b11.reshape(1, L).astype(f32),
        W12.astype(f32), b12.reshape(1, 1).astype(f32),
    )
    return out.reshape(1)


# speedup vs baseline: 12.5009x; 1.0261x over previous
import jax
import jax.numpy as jnp
from jax import lax
from jax.experimental import pallas as pl
from jax.experimental.pallas import tpu as pltpu
from jax.experimental.pallas import tpu_sc as plsc

N = 100000
E = 1600000
C_IN = 14
NC = 2
NS = 16
L = 16
CHUNK = 128
K_CH = 8
BLK = 2048
N_T = 100352
E_PAD = -(-E // (NC * NS * CHUNK * K_CH)) * (NC * NS * CHUNK * K_CH)
GRID = N_T // BLK


def _mesh():
    return plsc.VectorSubcoreMesh(
        core_axis_name="c", subcore_axis_name="s", num_cores=NC, num_subcores=NS
    )


_SC_PARAMS = pltpu.CompilerParams(use_tc_tiling_on_sc=False)


def _segsum(tab, srcp, dstp, zeros, feat_split):
    rpt = N_T // NS
    nch = (E_PAD // CHUNK) // NS if feat_split else (E_PAD // CHUNK) // (NC * NS)
    ngrp = nch // K_CH

    def body(tab_hbm, src_hbm, dst_hbm, z_hbm, out_hbm,
             idx_s, idx_d, rows, acc, sem_i, sem_g, sem_s):
        c = lax.axis_index("c")
        s = lax.axis_index("s")
        r0 = s * rpt
        pltpu.sync_copy(z_hbm.at[pl.ds(r0, rpt)], acc.at[pl.ds(r0, rpt)])
        plsc.subcore_barrier()
        cbase = s * nch if feat_split else (c * NS + s) * nch
        tab_off = c * N_T

        def group(g, carry):
            off = (cbase + g * K_CH) * CHUNK
            pltpu.sync_copy(src_hbm.at[pl.ds(off, K_CH * CHUNK)], idx_s)
            ids = [pltpu.async_copy(dst_hbm.at[pl.ds(off + j * CHUNK, CHUNK)],
                                    idx_d.at[j], sem_i) for j in range(K_CH)]
            if feat_split:
                for t in range(K_CH * CHUNK // L):
                    sl = pl.ds(t * L, L)
                    idx_s[sl] = idx_s[sl] + tab_off
            gds = [pltpu.async_copy(tab_hbm.at[idx_s.at[pl.ds(j * CHUNK, CHUNK)]],
                                    rows.at[j], sem_g) for j in range(K_CH)]
            for d in gds:
                d.wait()
            for d in ids:
                d.wait()
            sds = [pltpu.async_copy(rows.at[j], acc.at[idx_d.at[j]], sem_s, add=True)
                   for j in range(K_CH)]
            for d in sds:
                d.wait()
            return carry

        lax.fori_loop(0, ngrp, group, 0)
        plsc.subcore_barrier()
        pltpu.sync_copy(acc.at[pl.ds(r0, rpt)], out_hbm.at[pl.ds(c * N_T + r0, rpt)])

    return pl.kernel(
        body,
        out_type=jax.ShapeDtypeStruct((NC * N_T, L), jnp.float32),
        mesh=_mesh(),
        scratch_types=[
            pltpu.VMEM((K_CH * CHUNK,), jnp.int32),
            pltpu.VMEM((K_CH, CHUNK), jnp.int32),
            pltpu.VMEM((K_CH, CHUNK, L), jnp.float32),
            pltpu.VMEM_SHARED((N_T, L), jnp.float32),
            pltpu.SemaphoreType.DMA,
            pltpu.SemaphoreType.DMA,
            pltpu.SemaphoreType.DMA,
        ],
        compiler_params=_SC_PARAMS,
    )(tab, srcp, dstp, zeros)


def _dot(a, b):
    return jnp.dot(a, b, preferred_element_type=jnp.float32,
                   precision=lax.Precision.HIGHEST)


def _dense0(s0, xp, wl, wr):
    def body(p0_ref, p1_ref, x_ref, wl_ref, wr_ref, o_ref):
        agg = p0_ref[...] + p1_ref[...]
        o_ref[...] = jnp.maximum(
            _dot(agg, wl_ref[0]) + _dot(x_ref[...], wr_ref[0]), 0.0)

    return pl.pallas_call(
        body,
        grid=(NC, GRID),
        in_specs=[
            pl.BlockSpec((BLK, L), lambda h, i: (i, 0)),
            pl.BlockSpec((BLK, L), lambda h, i: (GRID + i, 0)),
            pl.BlockSpec((BLK, L), lambda h, i: (i, 0)),
            pl.BlockSpec((1, L, L), lambda h, i: (h, 0, 0)),
            pl.BlockSpec((1, L, L), lambda h, i: (h, 0, 0)),
        ],
        out_specs=pl.BlockSpec((BLK, L), lambda h, i: (h * GRID + i, 0)),
        out_shape=jax.ShapeDtypeStruct((NC * N_T, L), jnp.float32),
    )(s0, s0, xp, wl, wr)


def _dense1(s1, h0s, wla, wlb, wra, wrb, w11, b11, w12, b12):
    def body(a_ref, b_ref, ha_ref, hb_ref, wla_ref, wlb_ref, wra_ref, wrb_ref,
             w11_ref, b11_ref, w12_ref, b12_ref, o_ref, acc_ref):
        i = pl.program_id(0)
        h1 = jnp.maximum(
            _dot(a_ref[...], wla_ref[...]) + _dot(b_ref[...], wlb_ref[...])
            + _dot(ha_ref[...], wra_ref[...]) + _dot(hb_ref[...], wrb_ref[...]),
            0.0,
        )
        part = jnp.sum(h1, axis=0, keepdims=True)

        @pl.when(i == 0)
        def _():
            acc_ref[...] = part

        @pl.when(i > 0)
        def _():
            acc_ref[...] = acc_ref[...] + part

        @pl.when(i == GRID - 1)
        def _():
            z = jnp.maximum(_dot(acc_ref[...], w11_ref[...]) + b11_ref[...], 0.0)
            o_ref[...] = _dot(z, w12_ref[...]) + b12_ref[...]

    return pl.pallas_call(
        body,
        grid=(GRID,),
        in_specs=[
            pl.BlockSpec((BLK, L), lambda i: (i, 0)),
            pl.BlockSpec((BLK, L), lambda i: (GRID + i, 0)),
            pl.BlockSpec((BLK, L), lambda i: (i, 0)),
            pl.BlockSpec((BLK, L), lambda i: (GRID + i, 0)),
            pl.BlockSpec((L, 32), lambda i: (0, 0)),
            pl.BlockSpec((L, 32), lambda i: (0, 0)),
            pl.BlockSpec((L, 32), lambda i: (0, 0)),
            pl.BlockSpec((L, 32), lambda i: (0, 0)),
            pl.BlockSpec((32, L), lambda i: (0, 0)),
            pl.BlockSpec((1, L), lambda i: (0, 0)),
            pl.BlockSpec((L, 1), lambda i: (0, 0)),
            pl.BlockSpec((1, 1), lambda i: (0, 0)),
        ],
        out_specs=pl.BlockSpec((1, 1), lambda i: (0, 0)),
        out_shape=jax.ShapeDtypeStruct((1, 1), jnp.float32),
        scratch_shapes=[pltpu.VMEM((1, 32), jnp.float32)],
    )(s1, s1, h0s, h0s, wla, wlb, wra, wrb, w11, b11, w12, b12)


def kernel(x, edge_index, Wl0, Wr0, Wl1, Wr1, W11, b11, W12, b12):
    f32 = jnp.float32
    xp = jnp.zeros((N_T, L), f32).at[:N, :C_IN].set(x.astype(f32))
    ei = edge_index.astype(jnp.int32)
    srcp = jnp.pad(ei[0], (0, E_PAD - E), constant_values=N)
    dstp = jnp.pad(ei[1], (0, E_PAD - E), constant_values=N)
    zeros = jnp.zeros((N_T, L), f32)

    s0 = _segsum(xp, srcp, dstp, zeros, feat_split=False)
    wl0 = jnp.zeros((L, 32), f32).at[:C_IN].set(Wl0.astype(f32))
    wr0 = jnp.zeros((L, 32), f32).at[:C_IN].set(Wr0.astype(f32))
    wl3 = jnp.stack([wl0[:, :L], wl0[:, L:]])
    wr3 = jnp.stack([wr0[:, :L], wr0[:, L:]])
    h0s = _dense0(s0, xp, wl3, wr3)

    s1 = _segsum(h0s, srcp, dstp, zeros, feat_split=True)

    out = _dense1(
        s1, h0s,
        Wl1[:L].astype(f32), Wl1[L:].astype(f32),
        Wr1[:L].astype(f32), Wr1[L:].astype(f32),
        W11.astype(f32), b11.reshape(1, L).astype(f32),
        W12.astype(f32), b12.reshape(1, 1).astype(f32),
    )
    return out.reshape(1)


# speedup vs baseline: 19.9290x; 1.5942x over previous
import jax
import jax.numpy as jnp
from jax import lax
from jax.experimental import pallas as pl
from jax.experimental.pallas import tpu as pltpu
from jax.experimental.pallas import tpu_sc as plsc

N = 100000
E = 1600000
C_IN = 14
NC = 2
NS = 16
L = 16
CHUNK = 128
K_CH = 8
BLK = 2048
N_T = 100352
E_PAD = -(-E // (NC * NS * CHUNK * K_CH)) * (NC * NS * CHUNK * K_CH)
GRID = N_T // BLK


def _mesh():
    return plsc.VectorSubcoreMesh(
        core_axis_name="c", subcore_axis_name="s", num_cores=NC, num_subcores=NS
    )


_SC_PARAMS = pltpu.CompilerParams(use_tc_tiling_on_sc=False)


def _segsum(tab, srcp, dstp, zeros, feat_split):
    rpt = N_T // NS
    nch = (E_PAD // CHUNK) // NS if feat_split else (E_PAD // CHUNK) // (NC * NS)
    ngrp = nch // K_CH

    def body(tab_hbm, src_hbm, dst_hbm, z_hbm, out_hbm,
             idx_s, idx_d, rows, acc, sem_i, sem_g, sem_s):
        c = lax.axis_index("c")
        s = lax.axis_index("s")
        r0 = s * rpt
        pltpu.sync_copy(z_hbm.at[pl.ds(r0, rpt)], acc.at[pl.ds(r0, rpt)])
        plsc.subcore_barrier()
        cbase = s * nch if feat_split else (c * NS + s) * nch
        tab_off = c * N_T

        def group(g, carry):
            off = (cbase + g * K_CH) * CHUNK
            pltpu.sync_copy(src_hbm.at[pl.ds(off, K_CH * CHUNK)], idx_s)
            ids = [pltpu.async_copy(dst_hbm.at[pl.ds(off + j * CHUNK, CHUNK)],
                                    idx_d.at[j], sem_i) for j in range(K_CH)]
            if feat_split:
                for t in range(K_CH * CHUNK // L):
                    sl = pl.ds(t * L, L)
                    idx_s[sl] = idx_s[sl] + tab_off
            gds = [pltpu.async_copy(tab_hbm.at[idx_s.at[pl.ds(j * CHUNK, CHUNK)]],
                                    rows.at[j], sem_g) for j in range(K_CH)]
            for d in gds:
                d.wait()
            for d in ids:
                d.wait()
            sds = [pltpu.async_copy(rows.at[j], acc.at[idx_d.at[j]], sem_s, add=True)
                   for j in range(K_CH)]
            for d in sds:
                d.wait()
            return carry

        lax.fori_loop(0, ngrp, group, 0)
        plsc.subcore_barrier()
        pltpu.sync_copy(acc.at[pl.ds(r0, rpt)], out_hbm.at[pl.ds(c * N_T + r0, rpt)])

    return pl.kernel(
        body,
        out_type=jax.ShapeDtypeStruct((NC * N_T, L), jnp.float32),
        mesh=_mesh(),
        scratch_types=[
            pltpu.VMEM((K_CH * CHUNK,), jnp.int32),
            pltpu.VMEM((K_CH, CHUNK), jnp.int32),
            pltpu.VMEM((K_CH, CHUNK, L), jnp.float32),
            pltpu.VMEM_SHARED((N_T, L), jnp.float32),
            pltpu.SemaphoreType.DMA,
            pltpu.SemaphoreType.DMA,
            pltpu.SemaphoreType.DMA,
        ],
        compiler_params=_SC_PARAMS,
    )(tab, srcp, dstp, zeros)


def _dot(a, b):
    return jnp.dot(a, b, preferred_element_type=jnp.float32,
                   precision=lax.Precision.HIGHEST)


NW = N_T // 8
BLKW = NW // 8


def _bd(w16):
    return jnp.kron(jnp.eye(8, dtype=jnp.float32), w16)


def _dense0(s0w, xw, wlbd, wrbd):
    def body(p0_ref, p1_ref, x_ref, wl_ref, wr_ref, o_ref):
        agg = p0_ref[...] + p1_ref[...]
        o_ref[...] = jnp.maximum(
            _dot(agg, wl_ref[0]) + _dot(x_ref[...], wr_ref[0]), 0.0)

    return pl.pallas_call(
        body,
        grid=(NC, 8),
        in_specs=[
            pl.BlockSpec((BLKW, 128), lambda h, i: (i, 0)),
            pl.BlockSpec((BLKW, 128), lambda h, i: (8 + i, 0)),
            pl.BlockSpec((BLKW, 128), lambda h, i: (i, 0)),
            pl.BlockSpec((1, 128, 128), lambda h, i: (h, 0, 0)),
            pl.BlockSpec((1, 128, 128), lambda h, i: (h, 0, 0)),
        ],
        out_specs=pl.BlockSpec((BLKW, 128), lambda h, i: (h * 8 + i, 0)),
        out_shape=jax.ShapeDtypeStruct((NC * NW, 128), jnp.float32),
    )(s0w, s0w, xw, wlbd, wrbd)


def _dense1(s1w, h0w, wlabd, wlbbd, wrabd, wrbbd, fold, w11, b11, w12, b12):
    def body(a_ref, b_ref, ha_ref, hb_ref, wla_ref, wlb_ref, wra_ref, wrb_ref,
             f_ref, w11_ref, b11_ref, w12_ref, b12_ref, o_ref, acc_ref):
        h = pl.program_id(0)
        i = pl.program_id(1)
        h1 = jnp.maximum(
            _dot(a_ref[...], wla_ref[0]) + _dot(b_ref[...], wlb_ref[0])
            + _dot(ha_ref[...], wra_ref[0]) + _dot(hb_ref[...], wrb_ref[0]),
            0.0,
        )
        part = jnp.sum(h1, axis=0, keepdims=True)

        @pl.when(i == 0)
        def _():
            acc_ref[pl.ds(h, 1), :] = part

        @pl.when(i > 0)
        def _():
            acc_ref[pl.ds(h, 1), :] = acc_ref[pl.ds(h, 1), :] + part

        @pl.when((h == NC - 1) & (i == 7))
        def _():
            sa = _dot(acc_ref[0:1, :], f_ref[...])
            sb = _dot(acc_ref[1:2, :], f_ref[...])
            s32 = jnp.concatenate([sa, sb], axis=1)
            z = jnp.maximum(_dot(s32, w11_ref[...]) + b11_ref[...], 0.0)
            o_ref[...] = _dot(z, w12_ref[...]) + b12_ref[...]

    return pl.pallas_call(
        body,
        grid=(NC, 8),
        in_specs=[
            pl.BlockSpec((BLKW, 128), lambda h, i: (i, 0)),
            pl.BlockSpec((BLKW, 128), lambda h, i: (8 + i, 0)),
            pl.BlockSpec((BLKW, 128), lambda h, i: (i, 0)),
            pl.BlockSpec((BLKW, 128), lambda h, i: (8 + i, 0)),
            pl.BlockSpec((1, 128, 128), lambda h, i: (h, 0, 0)),
            pl.BlockSpec((1, 128, 128), lambda h, i: (h, 0, 0)),
            pl.BlockSpec((1, 128, 128), lambda h, i: (h, 0, 0)),
            pl.BlockSpec((1, 128, 128), lambda h, i: (h, 0, 0)),
            pl.BlockSpec((128, L), lambda h, i: (0, 0)),
            pl.BlockSpec((32, L), lambda h, i: (0, 0)),
            pl.BlockSpec((1, L), lambda h, i: (0, 0)),
            pl.BlockSpec((L, 1), lambda h, i: (0, 0)),
            pl.BlockSpec((1, 1), lambda h, i: (0, 0)),
        ],
        out_specs=pl.BlockSpec((1, 1), lambda h, i: (0, 0)),
        out_shape=jax.ShapeDtypeStruct((1, 1), jnp.float32),
        scratch_shapes=[pltpu.VMEM((NC, 128), jnp.float32)],
    )(s1w, s1w, h0w, h0w, wlabd, wlbbd, wrabd, wrbbd, fold, w11, b11, w12, b12)


def kernel(x, edge_index, Wl0, Wr0, Wl1, Wr1, W11, b11, W12, b12):
    f32 = jnp.float32
    xw = jnp.pad(x.astype(f32), ((0, N_T - N), (0, L - C_IN))).reshape(NW, 128)
    ei = edge_index.astype(jnp.int32)
    srcp = jnp.pad(ei[0], (0, E_PAD - E), constant_values=N)
    dstp = jnp.pad(ei[1], (0, E_PAD - E), constant_values=N)
    zeros = jnp.zeros((N_T, L), f32)

    s0 = _segsum(xw.reshape(N_T, L), srcp, dstp, zeros, feat_split=False)
    wl0 = jnp.zeros((L, 32), f32).at[:C_IN].set(Wl0.astype(f32))
    wr0 = jnp.zeros((L, 32), f32).at[:C_IN].set(Wr0.astype(f32))
    wlbd = jnp.stack([_bd(wl0[:, :L]), _bd(wl0[:, L:])])
    wrbd = jnp.stack([_bd(wr0[:, :L]), _bd(wr0[:, L:])])
    h0w = _dense0(s0.reshape(NC * NW, 128), xw, wlbd, wrbd)

    s1 = _segsum(h0w.reshape(NC * N_T, L), srcp, dstp, zeros, feat_split=True)

    Wl1f, Wr1f = Wl1.astype(f32), Wr1.astype(f32)
    wlabd = jnp.stack([_bd(Wl1f[:L, :L]), _bd(Wl1f[:L, L:])])
    wlbbd = jnp.stack([_bd(Wl1f[L:, :L]), _bd(Wl1f[L:, L:])])
    wrabd = jnp.stack([_bd(Wr1f[:L, :L]), _bd(Wr1f[:L, L:])])
    wrbbd = jnp.stack([_bd(Wr1f[L:, :L]), _bd(Wr1f[L:, L:])])
    fold = jnp.tile(jnp.eye(L, dtype=f32), (8, 1))

    out = _dense1(
        s1.reshape(NC * NW, 128), h0w,
        wlabd, wlbbd, wrabd, wrbbd, fold,
        W11.astype(f32), b11.reshape(1, L).astype(f32),
        W12.astype(f32), b12.reshape(1, 1).astype(f32),
    )
    return out.reshape(1)
